# Initial kernel scaffold; baseline (speedup 1.0000x reference)
import jax
import jax.numpy as jnp
from jax.experimental import pallas as pl


def kernel(tokens, table, prompt_embedding):
    raise NotImplementedError("write your pallas kernel here")


# speedup vs baseline: 2.5150x; 2.5150x over previous
import jax
import jax.numpy as jnp
from jax import lax
from jax.experimental import pallas as pl
from jax.experimental.pallas import tpu as pltpu
from jax.experimental.pallas import tpu_sc as plsc

VOCAB = 100000
DIM = 128
NUM_TOKENS = 20
BATCH = 1024
SEQ = 220
BODY = SEQ - NUM_TOKENS
HALF = BODY // 2

_info = plsc.get_sparse_core_info()
_NC, _NS = _info.num_cores, _info.num_subcores
NW = _NC * _NS
B_PER_W = BATCH // NW


def _sc_body(tok_hbm, table_hbm, prompt_hbm, out_hbm, idx_v, rows_v, prompt_v, sem):
    wid = lax.axis_index("s") * _NC + lax.axis_index("c")
    base = wid * B_PER_W
    pltpu.sync_copy(prompt_hbm, prompt_v)

    def body(i, carry):
        b = base + i
        pltpu.sync_copy(tok_hbm.at[b], idx_v)
        g0 = pltpu.async_copy(table_hbm.at[idx_v.at[0]], rows_v.at[pl.ds(0, HALF)], sem)
        g1 = pltpu.async_copy(table_hbm.at[idx_v.at[1]], rows_v.at[pl.ds(HALF, HALF)], sem)
        pltpu.sync_copy(prompt_v, out_hbm.at[b, pl.ds(0, NUM_TOKENS)])
        g0.wait()
        g1.wait()
        pltpu.sync_copy(rows_v, out_hbm.at[b, pl.ds(NUM_TOKENS, BODY)])
        return carry

    lax.fori_loop(0, B_PER_W, body, 0)


def kernel(tokens, table, prompt_embedding):
    tok = tokens[:, NUM_TOKENS:].astype(jnp.int32).reshape(BATCH, 2, HALF)
    sc = pl.kernel(
        _sc_body,
        out_type=jax.ShapeDtypeStruct((BATCH, SEQ, DIM), jnp.float32),
        mesh=plsc.VectorSubcoreMesh(core_axis_name="c", subcore_axis_name="s"),
        scratch_types=[
            pltpu.VMEM((2, HALF), jnp.int32),
            pltpu.VMEM((BODY, DIM), jnp.float32),
            pltpu.VMEM((NUM_TOKENS, DIM), jnp.float32),
            pltpu.SemaphoreType.DMA,
        ],
        compiler_params=pltpu.CompilerParams(use_tc_tiling_on_sc=False),
    )
    return sc(tok, table, prompt_embedding)


# speedup vs baseline: 2.8066x; 1.1159x over previous
import jax
import jax.numpy as jnp
from jax import lax
from jax.experimental import pallas as pl
from jax.experimental.pallas import tpu as pltpu
from jax.experimental.pallas import tpu_sc as plsc

VOCAB = 100000
DIM = 128
NUM_TOKENS = 20
BATCH = 1024
SEQ = 220
BODY = SEQ - NUM_TOKENS
HALF = BODY // 2

_info = plsc.get_sparse_core_info()
_NC, _NS = _info.num_cores, _info.num_subcores
NW = _NC * _NS
B_PER_W = BATCH // NW


def _sc_body(tok_hbm, table_hbm, prompt_hbm, out_hbm,
             idx_v, buf0, buf1, sem_g0, sem_g1, sem_w0, sem_w1):
    wid = lax.axis_index("s") * _NC + lax.axis_index("c")
    base = wid * B_PER_W

    bufs = (buf0, buf1)
    sems_g = (sem_g0, sem_g1)
    sems_w = (sem_w0, sem_w1)

    pltpu.sync_copy(tok_hbm.at[pl.ds(base, B_PER_W)], idx_v)
    pltpu.sync_copy(prompt_hbm, buf0.at[pl.ds(0, NUM_TOKENS)])
    pltpu.sync_copy(prompt_hbm, buf1.at[pl.ds(0, NUM_TOKENS)])

    def gather(i, k):
        g0 = pltpu.async_copy(table_hbm.at[idx_v.at[i, 0]],
                              bufs[k].at[pl.ds(NUM_TOKENS, HALF)], sems_g[k])
        g1 = pltpu.async_copy(table_hbm.at[idx_v.at[i, 1]],
                              bufs[k].at[pl.ds(NUM_TOKENS + HALF, HALF)], sems_g[k])
        return (g0, g1)

    pend_g = [gather(0, 0), None]
    pend_w = [None, None]
    for i in range(B_PER_W):
        k = i % 2
        nk = (i + 1) % 2
        if i + 1 < B_PER_W:
            if pend_w[nk] is not None:
                pend_w[nk].wait()
                pend_w[nk] = None
            pend_g[nk] = gather(i + 1, nk)
        pend_g[k][0].wait()
        pend_g[k][1].wait()
        pend_w[k] = pltpu.async_copy(bufs[k], out_hbm.at[base + i], sems_w[k])
    for k in range(2):
        if pend_w[k] is not None:
            pend_w[k].wait()


def kernel(tokens, table, prompt_embedding):
    tok = tokens[:, NUM_TOKENS:].astype(jnp.int32).reshape(BATCH, 2, HALF)
    sc = pl.kernel(
        _sc_body,
        out_type=jax.ShapeDtypeStruct((BATCH, SEQ, DIM), jnp.float32),
        mesh=plsc.VectorSubcoreMesh(core_axis_name="c", subcore_axis_name="s"),
        scratch_types=[
            pltpu.VMEM((B_PER_W, 2, HALF), jnp.int32),
            pltpu.VMEM((SEQ, DIM), jnp.float32),
            pltpu.VMEM((SEQ, DIM), jnp.float32),
            pltpu.SemaphoreType.DMA,
            pltpu.SemaphoreType.DMA,
            pltpu.SemaphoreType.DMA,
            pltpu.SemaphoreType.DMA,
        ],
        compiler_params=pltpu.CompilerParams(use_tc_tiling_on_sc=False),
    )
    return sc(tok, table, prompt_embedding)


# speedup vs baseline: 2.8170x; 1.0037x over previous
import jax
import jax.numpy as jnp
from jax import lax
from jax.experimental import pallas as pl
from jax.experimental.pallas import tpu as pltpu
from jax.experimental.pallas import tpu_sc as plsc

VOCAB = 100000
DIM = 128
NUM_TOKENS = 20
BATCH = 1024
SEQ = 220
BODY = SEQ - NUM_TOKENS
NBUF = 3

_info = plsc.get_sparse_core_info()
_NC, _NS = _info.num_cores, _info.num_subcores
NW = _NC * _NS
B_PER_W = BATCH // NW


def _sc_body(tok_hbm, table_hbm, prompt_hbm, out_hbm,
             idx_v, buf0, buf1, buf2,
             sem_g0, sem_g1, sem_g2, sem_w0, sem_w1, sem_w2):
    wid = lax.axis_index("s") * _NC + lax.axis_index("c")
    base = wid * B_PER_W

    bufs = (buf0, buf1, buf2)
    sems_g = (sem_g0, sem_g1, sem_g2)
    sems_w = (sem_w0, sem_w1, sem_w2)

    pltpu.sync_copy(tok_hbm.at[pl.ds(base, B_PER_W)], idx_v)
    for buf in bufs:
        pltpu.sync_copy(prompt_hbm, buf.at[pl.ds(0, NUM_TOKENS)])

    def gather(i, k):
        return pltpu.async_copy(table_hbm.at[idx_v.at[i]],
                                bufs[k].at[pl.ds(NUM_TOKENS, BODY)], sems_g[k])

    pend_g = [None] * NBUF
    pend_w = [None] * NBUF
    for i in range(NBUF - 1):
        pend_g[i] = gather(i, i)
    for i in range(B_PER_W):
        k = i % NBUF
        nk = (i + NBUF - 1) % NBUF
        if i + NBUF - 1 < B_PER_W:
            if pend_w[nk] is not None:
                pend_w[nk].wait()
                pend_w[nk] = None
            pend_g[nk] = gather(i + NBUF - 1, nk)
        pend_g[k].wait()
        pend_w[k] = pltpu.async_copy(bufs[k], out_hbm.at[base + i], sems_w[k])
    for k in range(NBUF):
        if pend_w[k] is not None:
            pend_w[k].wait()


def kernel(tokens, table, prompt_embedding):
    tok = tokens[:, NUM_TOKENS:].astype(jnp.int32)
    sc = pl.kernel(
        _sc_body,
        out_type=jax.ShapeDtypeStruct((BATCH, SEQ, DIM), jnp.float32),
        mesh=plsc.VectorSubcoreMesh(core_axis_name="c", subcore_axis_name="s"),
        scratch_types=[
            pltpu.VMEM((B_PER_W, BODY), jnp.int32),
            pltpu.VMEM((SEQ, DIM), jnp.float32),
            pltpu.VMEM((SEQ, DIM), jnp.float32),
            pltpu.VMEM((SEQ, DIM), jnp.float32),
            pltpu.SemaphoreType.DMA,
            pltpu.SemaphoreType.DMA,
            pltpu.SemaphoreType.DMA,
            pltpu.SemaphoreType.DMA,
            pltpu.SemaphoreType.DMA,
            pltpu.SemaphoreType.DMA,
        ],
        compiler_params=pltpu.CompilerParams(use_tc_tiling_on_sc=False),
    )
    return sc(tok, table, prompt_embedding)


# speedup vs baseline: 4.6639x; 1.6556x over previous
import jax
import jax.numpy as jnp
from jax import lax
from jax.experimental import pallas as pl
from jax.experimental.pallas import tpu as pltpu
from jax.experimental.pallas import tpu_sc as plsc

VOCAB = 100000
DIM = 128
NUM_TOKENS = 20
BATCH = 1024
SEQ = 220
BODY = SEQ - NUM_TOKENS
NBUF = 3

_info = plsc.get_sparse_core_info()
_NC, _NS = _info.num_cores, _info.num_subcores
NW = _NC * _NS
B_PER_W = BATCH // NW


def _sc_body(tok_hbm, table_hbm, prompt_hbm, out_hbm,
             idx_v, buf0, buf1, buf2,
             sem_g0, sem_g1, sem_g2, sem_w0, sem_w1, sem_w2):
    wid = lax.axis_index("s") * _NC + lax.axis_index("c")
    base = wid * B_PER_W

    bufs = (buf0, buf1, buf2)
    sems_g = (sem_g0, sem_g1, sem_g2)
    sems_w = (sem_w0, sem_w1, sem_w2)

    pltpu.sync_copy(tok_hbm.at[pl.ds(base * BODY, B_PER_W * BODY)], idx_v)
    for buf in bufs:
        pltpu.sync_copy(prompt_hbm, buf.at[pl.ds(0, NUM_TOKENS)])

    def gather(i, k):
        return pltpu.async_copy(table_hbm.at[idx_v.at[pl.ds(i * BODY, BODY)]],
                                bufs[k].at[pl.ds(NUM_TOKENS, BODY)], sems_g[k])

    pend_g = [None] * NBUF
    pend_w = [None] * NBUF
    for i in range(NBUF - 1):
        pend_g[i] = gather(i, i)
    for i in range(B_PER_W):
        k = i % NBUF
        nk = (i + NBUF - 1) % NBUF
        if i + NBUF - 1 < B_PER_W:
            if pend_w[nk] is not None:
                pend_w[nk].wait()
                pend_w[nk] = None
            pend_g[nk] = gather(i + NBUF - 1, nk)
        pend_g[k].wait()
        pend_w[k] = pltpu.async_copy(bufs[k], out_hbm.at[base + i], sems_w[k])
    for k in range(NBUF):
        if pend_w[k] is not None:
            pend_w[k].wait()


def kernel(tokens, table, prompt_embedding):
    tok = tokens[:, NUM_TOKENS:].astype(jnp.int32).reshape(-1)
    sc = pl.kernel(
        _sc_body,
        out_type=jax.ShapeDtypeStruct((BATCH, SEQ, DIM), jnp.float32),
        mesh=plsc.VectorSubcoreMesh(core_axis_name="c", subcore_axis_name="s"),
        scratch_types=[
            pltpu.VMEM((B_PER_W * BODY,), jnp.int32),
            pltpu.VMEM((SEQ, DIM), jnp.float32),
            pltpu.VMEM((SEQ, DIM), jnp.float32),
            pltpu.VMEM((SEQ, DIM), jnp.float32),
            pltpu.SemaphoreType.DMA,
            pltpu.SemaphoreType.DMA,
            pltpu.SemaphoreType.DMA,
            pltpu.SemaphoreType.DMA,
            pltpu.SemaphoreType.DMA,
            pltpu.SemaphoreType.DMA,
        ],
    )
    return sc(tok, table, prompt_embedding)
